# Initial kernel scaffold; baseline (speedup 1.0000x reference)
import jax
import jax.numpy as jnp
from jax.experimental import pallas as pl


def kernel(x):
    raise NotImplementedError("write your pallas kernel here")


# speedup vs baseline: 3.3722x; 3.3722x over previous
import functools
import math

import jax
import jax.numpy as jnp
from jax import lax
from jax.experimental import pallas as pl
from jax.experimental.pallas import tpu as pltpu
from jax.experimental.pallas import tpu_sc as plsc

_SPARSITY = 0.9
_NC = 2
_NS = 16
_NW = _NC * _NS
_LANES = 16


def _make(n_rows, row_len, r_chunk, unroll=8, interpret=False):
  assert n_rows % (_NW * r_chunk) == 0
  assert row_len % (_LANES * unroll) == 0
  rows_per_w = n_rows // _NW
  n_chunks = rows_per_w // r_chunk
  n_sparse = math.floor((1.0 - _SPARSITY) * row_len)
  k_rank = row_len - n_sparse - 1
  vregs_per_row = row_len // _LANES
  chunk_elems = r_chunk * row_len

  mesh = plsc.VectorSubcoreMesh(
      core_axis_name="c", subcore_axis_name="s",
      num_cores=_NC, num_subcores=_NS)

  @functools.partial(
      pl.kernel,
      out_type=jax.ShapeDtypeStruct((n_rows * row_len,), jnp.int32),
      mesh=mesh,
      scratch_types=[
          pltpu.VMEM((chunk_elems,), jnp.int32),
          pltpu.VMEM((chunk_elems,), jnp.int32),
      ],
      compiler_params=pltpu.CompilerParams(needs_layout_passes=False),
      interpret=interpret,
  )
  def sc_kernel(x_hbm, o_hbm, xbuf, abuf):
    wid = lax.axis_index("s") * _NC + lax.axis_index("c")
    wbase = wid * rows_per_w * row_len

    def chunk_body(ci, _):
      off = wbase + ci * chunk_elems
      pltpu.sync_copy(x_hbm.at[pl.ds(off, chunk_elems)], xbuf)

      def abs_body(j, _):
        for u in range(unroll):
          o2 = (j * unroll + u) * _LANES
          v = xbuf[pl.ds(o2, _LANES)]
          abuf[pl.ds(o2, _LANES)] = v & jnp.int32(0x7FFFFFFF)
        return 0
      lax.fori_loop(0, chunk_elems // (_LANES * unroll), abs_body, 0,
                    unroll=False)

      def row_body(r, _):
        rbase = r * row_len

        ones_v = jnp.full((_LANES,), 1, jnp.int32)
        k_v = jnp.full((_LANES,), k_rank, jnp.int32)

        def bit_body(i, res):
          cand = res | jnp.left_shift(ones_v, 30 - i)

          def cnt_body(j, acc):
            for u in range(unroll):
              a = abuf[pl.ds(rbase + (j * unroll + u) * _LANES, _LANES)]
              acc = acc + plsc.all_reduce_population_count(a < cand)
            return acc
          cnt = lax.fori_loop(0, vregs_per_row // unroll, cnt_body,
                              jnp.zeros((_LANES,), jnp.int32),
                              unroll=False)
          return jnp.where(cnt <= k_v, cand, res)

        res = lax.fori_loop(0, 31, bit_body,
                            jnp.zeros((_LANES,), jnp.int32), unroll=False)

        def mask_body(j, _):
          for u in range(unroll):
            o2 = rbase + (j * unroll + u) * _LANES
            a = abuf[pl.ds(o2, _LANES)]
            v = xbuf[pl.ds(o2, _LANES)]
            xbuf[pl.ds(o2, _LANES)] = jnp.where(a >= res, v, jnp.int32(0))
          return 0
        lax.fori_loop(0, vregs_per_row // unroll, mask_body, 0,
                      unroll=False)
        return 0

      lax.fori_loop(0, r_chunk, row_body, 0, unroll=False)
      pltpu.sync_copy(xbuf, o_hbm.at[pl.ds(off, chunk_elems)])
      return 0

    lax.fori_loop(0, n_chunks, chunk_body, 0, unroll=False)

  return sc_kernel


def kernel(x):
  shape = x.shape
  row_len = shape[-1]
  n_rows = x.size // row_len
  sc_kernel = _make(n_rows, row_len, r_chunk=8)
  x_bits = lax.bitcast_convert_type(x, jnp.int32).reshape(-1)
  out_bits = sc_kernel(x_bits)
  return lax.bitcast_convert_type(out_bits.reshape(shape), jnp.float32)
